# Initial kernel scaffold; baseline (speedup 1.0000x reference)
import jax
import jax.numpy as jnp
from jax.experimental import pallas as pl


def kernel(batch_points, field):
    raise NotImplementedError("write your pallas kernel here")


# speedup vs baseline: 378.7553x; 378.7553x over previous
import functools

import jax
import jax.numpy as jnp
from jax.experimental import pallas as pl

N = 2048
B = 16
K = 32
C = 16
GS = 32
BN1 = 128
BN2 = 32
BIG = 3.0e38


def _knn_kernel(q_ref, ptsT_ref, out_ref):
    q = q_ref[0]
    ptsT = ptsT_ref[0]
    inner = jax.lax.dot_general(q, ptsT, (((1,), (0,)), ((), ())),
                                preferred_element_type=jnp.float32)
    xx_q = jnp.sum(q * q, axis=1, keepdims=True)
    xx_all = jnp.sum(ptsT * ptsT, axis=0, keepdims=True)
    dist = xx_q - 2.0 * inner + xx_all

    lane = jax.lax.broadcasted_iota(jnp.int32, (BN1, N), 1)
    rels = []
    d = dist
    for _ in range(K):
        val = jnp.min(d, axis=1, keepdims=True)
        eq = d <= val
        sel = jnp.min(jnp.where(eq, lane, N + 1), axis=1, keepdims=True)
        hit = lane == sel
        onehot = hit.astype(jnp.float32)
        coords = jax.lax.dot_general(onehot, ptsT, (((1,), (1,)), ((), ())),
                                     preferred_element_type=jnp.float32)
        rels.append(coords - q)
        d = jnp.where(hit, BIG, d)

    allrel = jnp.concatenate(rels, axis=1)
    gmin = jnp.min(allrel)
    gmax = jnp.max(allrel)
    ones = jnp.ones((BN1, 1), jnp.float32)
    aux = jnp.concatenate(
        [ones * gmin, ones * gmax, jnp.zeros((BN1, 3), jnp.float32)], axis=1)
    for t in range(K):
        out_ref[0, t, :, :] = jnp.concatenate([rels[t], aux], axis=1)


def _sample_kernel(rel_ref, zoom_ref, field_ref, out_ref):
    inv = 1.0 / zoom_ref[0, 0]
    rel = rel_ref[0].reshape(K * BN2, 8)
    P = K * BN2

    def unnorm(col):
        g = rel[:, col:col + 1] * inv
        return jnp.clip(((g + 1.0) * GS - 1.0) * 0.5, 0.0, GS - 1.0)

    ix, iy, iz = unnorm(0), unnorm(1), unnorm(2)

    def split(i):
        i0f = jnp.floor(i)
        w = i - i0f
        i0 = jnp.clip(i0f.astype(jnp.int32), 0, GS - 1)
        i1 = jnp.clip(i0 + 1, 0, GS - 1)
        return i0, i1, w

    ix0, ix1, wx = split(ix)
    iy0, iy1, wy = split(iy)
    iz0, iz1, wz = split(iz)

    r00 = iz0 * GS + iy0
    r01 = iz0 * GS + iy1
    r10 = iz1 * GS + iy0
    r11 = iz1 * GS + iy1
    w00 = (1.0 - wz) * (1.0 - wy)
    w01 = (1.0 - wz) * wy
    w10 = wz * (1.0 - wy)
    w11 = wz * wy

    iota_r = jax.lax.broadcasted_iota(jnp.int32, (P, GS * GS), 1)
    wzy = ((iota_r == r00).astype(jnp.float32) * w00
           + (iota_r == r01).astype(jnp.float32) * w01
           + (iota_r == r10).astype(jnp.float32) * w10
           + (iota_r == r11).astype(jnp.float32) * w11)

    g = jax.lax.dot_general(wzy, field_ref[...], (((1,), (0,)), ((), ())),
                            preferred_element_type=jnp.float32)

    xcol = jax.lax.broadcasted_iota(jnp.int32, (P, GS * C), 1) // C
    wx_exp = ((xcol == ix0).astype(jnp.float32) * (1.0 - wx)
              + (xcol == ix1).astype(jnp.float32) * wx)
    prod = g * wx_exp

    ri = jax.lax.broadcasted_iota(jnp.int32, (GS * C, C), 0)
    ci = jax.lax.broadcasted_iota(jnp.int32, (GS * C, C), 1)
    s = (ri % C == ci).astype(jnp.float32)
    vals = jax.lax.dot_general(prod, s, (((1,), (0,)), ((), ())),
                               preferred_element_type=jnp.float32)

    out_ref[0] = jnp.max(vals.reshape(K, BN2, C), axis=0)


@jax.jit
def kernel(batch_points, field):
    ptsT = jnp.swapaxes(batch_points, 1, 2)

    rel = pl.pallas_call(
        _knn_kernel,
        grid=(B, N // BN1),
        in_specs=[
            pl.BlockSpec((1, BN1, 3), lambda b, n: (b, n, 0)),
            pl.BlockSpec((1, 3, N), lambda b, n: (b, 0, 0)),
        ],
        out_specs=pl.BlockSpec((1, K, BN1, 8), lambda b, n: (b, 0, n, 0)),
        out_shape=jax.ShapeDtypeStruct((B, K, N, 8), jnp.float32),
    )(batch_points, ptsT)

    gmin = jnp.min(rel[:, 0, :, 3])
    gmax = jnp.max(rel[:, 0, :, 4])
    zoom = jnp.maximum(jnp.abs(gmin), jnp.abs(gmax)).reshape(1, 1)

    field_r = jnp.transpose(field[0], (1, 2, 3, 0)).reshape(GS * GS, GS * C)

    out = pl.pallas_call(
        _sample_kernel,
        grid=(B, N // BN2),
        in_specs=[
            pl.BlockSpec((1, K, BN2, 8), lambda b, n: (b, 0, n, 0)),
            pl.BlockSpec((1, 1), lambda b, n: (0, 0)),
            pl.BlockSpec((GS * GS, GS * C), lambda b, n: (0, 0)),
        ],
        out_specs=pl.BlockSpec((1, BN2, C), lambda b, n: (b, n, 0)),
        out_shape=jax.ShapeDtypeStruct((B, N, C), jnp.float32),
    )(rel, zoom, field_r)

    return jnp.swapaxes(out, 1, 2)
